# baseline (speedup 1.0000x reference)
import functools

import numpy as np
import jax
import jax.numpy as jnp
from jax import lax
from jax.experimental import pallas as pl
from jax.experimental.pallas import tpu as pltpu
from jax.experimental.pallas import tpu_sc as plsc

N = 10000
E = 320000
D = 128
C_U = 1.0
C_SIGMA = 2.0
COEF = float(np.sqrt(C_SIGMA / D))

NC, NS = 2, 16
NW = NC * NS
CHUNK = 128
N_PAD = 10240
SLAB = N_PAD // NS
NCHUNK = 80
TILE_E = NCHUNK * CHUNK
E_PAD = NW * TILE_E
ZROWS = 128

_MESH = plsc.VectorSubcoreMesh(core_axis_name="c", subcore_axis_name="s")


@functools.partial(
    pl.kernel,
    out_type=jax.ShapeDtypeStruct((NC, N_PAD), jnp.float32),
    mesh=_MESH,
    scratch_types=[
        pltpu.VMEM_SHARED((N_PAD,), jnp.float32),
        pltpu.VMEM((NCHUNK, CHUNK), jnp.int32),
        pltpu.VMEM((CHUNK,), jnp.float32),
        pltpu.VMEM((SLAB,), jnp.float32),
    ],
)
def _deg_kernel(row_hbm, deg_out, deg_sh, row_v, ones_v, zb_v):
    cid = lax.axis_index("c")
    sid = lax.axis_index("s")
    wid = cid * NS + sid

    def fill_ones(i, _):
        ones_v[pl.ds(i * 16, 16)] = jnp.ones((16,), jnp.float32)
        return 0

    lax.fori_loop(0, CHUNK // 16, fill_ones, 0)

    def fill_zero(i, _):
        zb_v[pl.ds(i * 16, 16)] = jnp.zeros((16,), jnp.float32)
        return 0

    lax.fori_loop(0, SLAB // 16, fill_zero, 0)
    pltpu.sync_copy(zb_v, deg_sh.at[pl.ds(sid * SLAB, SLAB)])
    pltpu.sync_copy(row_hbm.at[pl.ds(wid * NCHUNK, NCHUNK)], row_v)
    plsc.subcore_barrier()

    def chunk(i, _):
        pltpu.sync_copy(ones_v, deg_sh.at[row_v.at[i]], add=True)
        return 0

    lax.fori_loop(0, NCHUNK, chunk, 0)
    plsc.subcore_barrier()
    pltpu.sync_copy(deg_sh.at[pl.ds(sid * SLAB, SLAB)],
                    deg_out.at[cid, pl.ds(sid * SLAB, SLAB)])


GROUP = 8
NGROUP = NCHUNK // GROUP


@functools.partial(
    pl.kernel,
    out_type=jax.ShapeDtypeStruct((NC, N_PAD, D), jnp.float32),
    mesh=_MESH,
    scratch_types=[
        pltpu.VMEM_SHARED((N_PAD, D), jnp.float32),
        pltpu.VMEM((GROUP, CHUNK), jnp.int32),
        pltpu.VMEM((GROUP, CHUNK), jnp.int32),
        pltpu.VMEM((CHUNK, D), jnp.float32),
        pltpu.VMEM((CHUNK, D), jnp.float32),
        pltpu.SemaphoreType.DMA,
        pltpu.SemaphoreType.DMA,
    ],
)
def _scatter_kernel(g_hbm, row_hbm, col_hbm, z_hbm, s_out,
                    acc_sh, row_v, col_v, buf_a, buf_b, sem_a, sem_b):
    cid = lax.axis_index("c")
    sid = lax.axis_index("s")
    wid = cid * NS + sid

    for k in range(SLAB // ZROWS):
        pltpu.sync_copy(z_hbm, acc_sh.at[pl.ds(sid * SLAB + k * ZROWS, ZROWS)])
    plsc.subcore_barrier()

    bufs = (buf_a, buf_b)
    sems = (sem_a, sem_b)

    def group(gi, _):
        blk = wid * NCHUNK + gi * GROUP
        pltpu.sync_copy(row_hbm.at[pl.ds(blk, GROUP)], row_v)
        pltpu.sync_copy(col_hbm.at[pl.ds(blk, GROUP)], col_v)
        pltpu.async_copy(g_hbm.at[row_v.at[0]], buf_a, sem_a)
        for k in range(GROUP - 1):
            pltpu.async_copy(g_hbm.at[row_v.at[k + 1]],
                             bufs[(k + 1) % 2], sems[(k + 1) % 2])
            pltpu.make_async_copy(g_hbm.at[pl.ds(0, CHUNK)],
                                  bufs[k % 2], sems[k % 2]).wait()
            pltpu.sync_copy(bufs[k % 2], acc_sh.at[col_v.at[k]], add=True)
        pltpu.make_async_copy(g_hbm.at[pl.ds(0, CHUNK)],
                              bufs[(GROUP - 1) % 2], sems[(GROUP - 1) % 2]).wait()
        pltpu.sync_copy(bufs[(GROUP - 1) % 2],
                        acc_sh.at[col_v.at[GROUP - 1]], add=True)
        return 0

    lax.fori_loop(0, NGROUP, group, 0)
    plsc.subcore_barrier()
    pltpu.sync_copy(acc_sh.at[pl.ds(sid * SLAB, SLAB)],
                    s_out.at[cid, pl.ds(sid * SLAB, SLAB)])


_BM = 256


def _lin_body(x_ref, wt_ref, b_ref, da_ref, db_ref, g_ref, dinv_ref):
    deg = da_ref[...] + db_ref[...] + 1.0
    dinv = lax.rsqrt(deg)
    h = jnp.dot(x_ref[...], wt_ref[...], preferred_element_type=jnp.float32)
    g_ref[...] = dinv * (h + b_ref[...])
    dinv_ref[...] = dinv


_lin_call = pl.pallas_call(
    _lin_body,
    grid=(N_PAD // _BM,),
    in_specs=[
        pl.BlockSpec((_BM, D), lambda i: (i, 0)),
        pl.BlockSpec((D, D), lambda i: (0, 0)),
        pl.BlockSpec((1, D), lambda i: (0, 0)),
        pl.BlockSpec((_BM, 1), lambda i: (i, 0)),
        pl.BlockSpec((_BM, 1), lambda i: (i, 0)),
    ],
    out_specs=[
        pl.BlockSpec((_BM, D), lambda i: (i, 0)),
        pl.BlockSpec((_BM, 1), lambda i: (i, 0)),
    ],
    out_shape=[
        jax.ShapeDtypeStruct((N_PAD, D), jnp.float32),
        jax.ShapeDtypeStruct((N_PAD, 1), jnp.float32),
    ],
)


def _fin_body(s0_ref, s1_ref, g_ref, dinv_ref, o_ref):
    s = s0_ref[0] + s1_ref[0] + g_ref[...]
    o_ref[...] = COEF * jnp.maximum(C_U * dinv_ref[...] * s, 0.0)


_fin_call = pl.pallas_call(
    _fin_body,
    grid=(N_PAD // _BM,),
    in_specs=[
        pl.BlockSpec((1, _BM, D), lambda i: (0, i, 0)),
        pl.BlockSpec((1, _BM, D), lambda i: (1, i, 0)),
        pl.BlockSpec((_BM, D), lambda i: (i, 0)),
        pl.BlockSpec((_BM, 1), lambda i: (i, 0)),
    ],
    out_specs=pl.BlockSpec((_BM, D), lambda i: (i, 0)),
    out_shape=jax.ShapeDtypeStruct((N, D), jnp.float32),
)


def kernel(x, edge_index, W, b):
    row = edge_index[0]
    col = edge_index[1]
    pad = jnp.full((E_PAD - E,), N_PAD - 1, dtype=jnp.int32)
    row_pad = jnp.concatenate([row, pad]).reshape(NW * NCHUNK, CHUNK)
    col_pad = jnp.concatenate([col, pad]).reshape(NW * NCHUNK, CHUNK)
    x_pad = jnp.pad(x, ((0, N_PAD - N), (0, 0)))

    degp = _deg_kernel(row_pad)
    da = degp[0].reshape(N_PAD, 1)
    db = degp[1].reshape(N_PAD, 1)
    g, dinv = _lin_call(x_pad, W.T, b.reshape(1, D), da, db)

    zeros = jnp.zeros((ZROWS, D), jnp.float32)
    S = _scatter_kernel(g, row_pad, col_pad, zeros)
    return _fin_call(S, S, g, dinv)


# speedup vs baseline: 2.8220x; 2.8220x over previous
import functools

import numpy as np
import jax
import jax.numpy as jnp
from jax import lax
from jax.experimental import pallas as pl
from jax.experimental.pallas import tpu as pltpu
from jax.experimental.pallas import tpu_sc as plsc

N = 10000
E = 320000
D = 128
C_U = 1.0
C_SIGMA = 2.0
COEF = float(np.sqrt(C_SIGMA / D))

NC, NS = 2, 16
NW = NC * NS
CHUNK = 128
N_PAD = 10240
SLAB = N_PAD // NS
NCHUNK = 80
TILE_E = NCHUNK * CHUNK
E_PAD = NW * TILE_E
ZROWS = 128

_MESH = plsc.VectorSubcoreMesh(core_axis_name="c", subcore_axis_name="s")


@functools.partial(
    pl.kernel,
    out_type=jax.ShapeDtypeStruct((NC, N_PAD), jnp.float32),
    mesh=_MESH,
    scratch_types=[
        pltpu.VMEM_SHARED((N_PAD,), jnp.float32),
        pltpu.VMEM((NCHUNK, CHUNK), jnp.int32),
        pltpu.VMEM((CHUNK,), jnp.float32),
        pltpu.VMEM((SLAB,), jnp.float32),
    ],
)
def _deg_kernel(row_hbm, deg_out, deg_sh, row_v, ones_v, zb_v):
    cid = lax.axis_index("c")
    sid = lax.axis_index("s")
    wid = cid * NS + sid

    def fill_ones(i, _):
        ones_v[pl.ds(i * 16, 16)] = jnp.ones((16,), jnp.float32)
        return 0

    lax.fori_loop(0, CHUNK // 16, fill_ones, 0)

    def fill_zero(i, _):
        zb_v[pl.ds(i * 16, 16)] = jnp.zeros((16,), jnp.float32)
        return 0

    lax.fori_loop(0, SLAB // 16, fill_zero, 0)
    pltpu.sync_copy(zb_v, deg_sh.at[pl.ds(sid * SLAB, SLAB)])
    pltpu.sync_copy(row_hbm.at[pl.ds(wid * NCHUNK, NCHUNK)], row_v)
    plsc.subcore_barrier()

    def chunk(i, _):
        pltpu.sync_copy(ones_v, deg_sh.at[row_v.at[i]], add=True)
        return 0

    lax.fori_loop(0, NCHUNK, chunk, 0)
    plsc.subcore_barrier()
    pltpu.sync_copy(deg_sh.at[pl.ds(sid * SLAB, SLAB)],
                    deg_out.at[cid, pl.ds(sid * SLAB, SLAB)])


GROUP = 8
NGROUP = NCHUNK // GROUP


@functools.partial(
    pl.kernel,
    out_type=jax.ShapeDtypeStruct((NC, N_PAD, D), jnp.float32),
    mesh=_MESH,
    scratch_types=[
        pltpu.VMEM_SHARED((N_PAD, D), jnp.float32),
        pltpu.VMEM((GROUP, CHUNK), jnp.int32),
        pltpu.VMEM((GROUP, CHUNK), jnp.int32),
        pltpu.VMEM((CHUNK, D), jnp.float32),
        pltpu.VMEM((CHUNK, D), jnp.float32),
        pltpu.SemaphoreType.DMA,
        pltpu.SemaphoreType.DMA,
    ],
)
def _scatter_kernel(g_hbm, row_hbm, col_hbm, z_hbm, s_out,
                    acc_sh, row_v, col_v, buf_a, buf_b, sem_a, sem_b):
    cid = lax.axis_index("c")
    sid = lax.axis_index("s")
    wid = cid * NS + sid

    for k in range(SLAB // ZROWS):
        pltpu.sync_copy(z_hbm, acc_sh.at[pl.ds(sid * SLAB + k * ZROWS, ZROWS)])
    plsc.subcore_barrier()

    bufs = (buf_a, buf_b)
    sems = (sem_a, sem_b)

    def group(gi, _):
        blk = wid * NCHUNK + gi * GROUP
        pltpu.sync_copy(row_hbm.at[pl.ds(blk, GROUP)], row_v)
        pltpu.sync_copy(col_hbm.at[pl.ds(blk, GROUP)], col_v)
        pltpu.async_copy(g_hbm.at[row_v.at[0]], buf_a, sem_a)
        for k in range(GROUP - 1):
            pltpu.async_copy(g_hbm.at[row_v.at[k + 1]],
                             bufs[(k + 1) % 2], sems[(k + 1) % 2])
            pltpu.make_async_copy(g_hbm.at[pl.ds(0, CHUNK)],
                                  bufs[k % 2], sems[k % 2]).wait()
            pltpu.sync_copy(bufs[k % 2], acc_sh.at[col_v.at[k]], add=True)
        pltpu.make_async_copy(g_hbm.at[pl.ds(0, CHUNK)],
                              bufs[(GROUP - 1) % 2], sems[(GROUP - 1) % 2]).wait()
        pltpu.sync_copy(bufs[(GROUP - 1) % 2],
                        acc_sh.at[col_v.at[GROUP - 1]], add=True)
        return 0

    lax.fori_loop(0, NGROUP, group, 0)
    plsc.subcore_barrier()
    pltpu.sync_copy(acc_sh.at[pl.ds(sid * SLAB, SLAB)],
                    s_out.at[cid, pl.ds(sid * SLAB, SLAB)])


_BM = 256


def _lin_body(x_ref, wt_ref, b_ref, da_ref, db_ref, g_ref, dinv_ref):
    deg = da_ref[...] + db_ref[...] + 1.0
    dinv = lax.rsqrt(deg)
    h = jnp.dot(x_ref[...], wt_ref[...], preferred_element_type=jnp.float32)
    g_ref[...] = dinv * (h + b_ref[...])
    dinv_ref[...] = dinv


_lin_call = pl.pallas_call(
    _lin_body,
    grid=(N_PAD // _BM,),
    in_specs=[
        pl.BlockSpec((_BM, D), lambda i: (i, 0)),
        pl.BlockSpec((D, D), lambda i: (0, 0)),
        pl.BlockSpec((1, D), lambda i: (0, 0)),
        pl.BlockSpec((_BM, 1), lambda i: (i, 0)),
        pl.BlockSpec((_BM, 1), lambda i: (i, 0)),
    ],
    out_specs=[
        pl.BlockSpec((_BM, D), lambda i: (i, 0)),
        pl.BlockSpec((_BM, 1), lambda i: (i, 0)),
    ],
    out_shape=[
        jax.ShapeDtypeStruct((N_PAD, D), jnp.float32),
        jax.ShapeDtypeStruct((N_PAD, 1), jnp.float32),
    ],
)


def _fin_body(s0_ref, s1_ref, g_ref, dinv_ref, o_ref):
    s = s0_ref[0] + s1_ref[0] + g_ref[...]
    o_ref[...] = COEF * jnp.maximum(C_U * dinv_ref[...] * s, 0.0)


_fin_call = pl.pallas_call(
    _fin_body,
    grid=(N_PAD // _BM,),
    in_specs=[
        pl.BlockSpec((1, _BM, D), lambda i: (0, i, 0)),
        pl.BlockSpec((1, _BM, D), lambda i: (1, i, 0)),
        pl.BlockSpec((_BM, D), lambda i: (i, 0)),
        pl.BlockSpec((_BM, 1), lambda i: (i, 0)),
    ],
    out_specs=pl.BlockSpec((_BM, D), lambda i: (i, 0)),
    out_shape=jax.ShapeDtypeStruct((N, D), jnp.float32),
)


def kernel(x, edge_index, W, b):
    row = edge_index[0]
    col = edge_index[1]
    pad = (N + jnp.arange(E_PAD - E, dtype=jnp.int32) % (N_PAD - N))
    row_pad = jnp.concatenate([row, pad]).reshape(NW * NCHUNK, CHUNK)
    col_pad = jnp.concatenate([col, pad]).reshape(NW * NCHUNK, CHUNK)
    x_pad = jnp.pad(x, ((0, N_PAD - N), (0, 0)))

    degp = _deg_kernel(row_pad)
    da = degp[0].reshape(N_PAD, 1)
    db = degp[1].reshape(N_PAD, 1)
    g, dinv = _lin_call(x_pad, W.T, b.reshape(1, D), da, db)

    zeros = jnp.zeros((ZROWS, D), jnp.float32)
    S = _scatter_kernel(g, row_pad, col_pad, zeros)
    return _fin_call(S, S, g, dinv)


# speedup vs baseline: 3.4454x; 1.2209x over previous
import functools

import numpy as np
import jax
import jax.numpy as jnp
from jax import lax
from jax.experimental import pallas as pl
from jax.experimental.pallas import tpu as pltpu
from jax.experimental.pallas import tpu_sc as plsc

N = 10000
E = 320000
D = 128
C_U = 1.0
C_SIGMA = 2.0
COEF = float(np.sqrt(C_SIGMA / D))

NC, NS = 2, 16
NW = NC * NS
CHUNK = 128
N_PAD = 10240
SLAB = N_PAD // NS
NCHUNK = 80
TILE_E = NCHUNK * CHUNK
E_PAD = NW * TILE_E
ZROWS = 128

_MESH = plsc.VectorSubcoreMesh(core_axis_name="c", subcore_axis_name="s")


@functools.partial(
    pl.kernel,
    out_type=jax.ShapeDtypeStruct((NC, N_PAD), jnp.float32),
    mesh=_MESH,
    scratch_types=[
        pltpu.VMEM_SHARED((N_PAD,), jnp.float32),
        pltpu.VMEM((NCHUNK, CHUNK), jnp.int32),
        pltpu.VMEM((CHUNK,), jnp.float32),
        pltpu.VMEM((SLAB,), jnp.float32),
    ],
)
def _deg_kernel(row_hbm, deg_out, deg_sh, row_v, ones_v, zb_v):
    cid = lax.axis_index("c")
    sid = lax.axis_index("s")
    wid = cid * NS + sid

    def fill_ones(i, _):
        ones_v[pl.ds(i * 16, 16)] = jnp.ones((16,), jnp.float32)
        return 0

    lax.fori_loop(0, CHUNK // 16, fill_ones, 0)

    def fill_zero(i, _):
        zb_v[pl.ds(i * 16, 16)] = jnp.zeros((16,), jnp.float32)
        return 0

    lax.fori_loop(0, SLAB // 16, fill_zero, 0)
    pltpu.sync_copy(zb_v, deg_sh.at[pl.ds(sid * SLAB, SLAB)])
    pltpu.sync_copy(row_hbm.at[pl.ds(wid * NCHUNK, NCHUNK)], row_v)
    plsc.subcore_barrier()

    def chunk(i, _):
        pltpu.sync_copy(ones_v, deg_sh.at[row_v.at[i]], add=True)
        return 0

    lax.fori_loop(0, NCHUNK, chunk, 0)
    plsc.subcore_barrier()
    pltpu.sync_copy(deg_sh.at[pl.ds(sid * SLAB, SLAB)],
                    deg_out.at[cid, pl.ds(sid * SLAB, SLAB)])


GROUP = 8
NGROUP = NCHUNK // GROUP


@functools.partial(
    pl.kernel,
    out_type=jax.ShapeDtypeStruct((NC, N_PAD, D), jnp.float32),
    mesh=_MESH,
    scratch_types=[
        pltpu.VMEM_SHARED((N_PAD, D), jnp.float32),
        pltpu.VMEM((GROUP, CHUNK), jnp.int32),
        pltpu.VMEM((GROUP, CHUNK), jnp.int32),
        pltpu.VMEM((CHUNK, D), jnp.float32),
        pltpu.VMEM((CHUNK, D), jnp.float32),
        pltpu.SemaphoreType.DMA,
        pltpu.SemaphoreType.DMA,
    ],
)
def _scatter_kernel(g_hbm, row_hbm, col_hbm, z_hbm, s_out,
                    acc_sh, row_v, col_v, buf_a, buf_b, sem_a, sem_b):
    cid = lax.axis_index("c")
    sid = lax.axis_index("s")
    wid = cid * NS + sid

    for k in range(SLAB // ZROWS):
        pltpu.sync_copy(z_hbm, acc_sh.at[pl.ds(sid * SLAB + k * ZROWS, ZROWS)])
    plsc.subcore_barrier()

    bufs = (buf_a, buf_b)
    sems = (sem_a, sem_b)

    def group(gi, _):
        blk = wid * NCHUNK + gi * GROUP
        pltpu.sync_copy(row_hbm.at[pl.ds(blk, GROUP)], row_v)
        pltpu.sync_copy(col_hbm.at[pl.ds(blk, GROUP)], col_v)
        pltpu.async_copy(g_hbm.at[row_v.at[0]], buf_a, sem_a)
        for k in range(GROUP - 1):
            pltpu.async_copy(g_hbm.at[row_v.at[k + 1]],
                             bufs[(k + 1) % 2], sems[(k + 1) % 2])
            pltpu.make_async_copy(g_hbm.at[pl.ds(0, CHUNK)],
                                  bufs[k % 2], sems[k % 2]).wait()
            pltpu.sync_copy(bufs[k % 2], acc_sh.at[col_v.at[k]], add=True)
        pltpu.make_async_copy(g_hbm.at[pl.ds(0, CHUNK)],
                              bufs[(GROUP - 1) % 2], sems[(GROUP - 1) % 2]).wait()
        pltpu.sync_copy(bufs[(GROUP - 1) % 2],
                        acc_sh.at[col_v.at[GROUP - 1]], add=True)
        return 0

    lax.fori_loop(0, NGROUP, group, 0)
    plsc.subcore_barrier()
    pltpu.sync_copy(acc_sh.at[pl.ds(sid * SLAB, SLAB)],
                    s_out.at[cid, pl.ds(sid * SLAB, SLAB)])


def _lin_body(x_ref, w_ref, b_ref, dp_ref, g_ref, dinv_ref):
    deg = (dp_ref[0] + dp_ref[1] + 1.0).reshape(N_PAD, 1)
    dinv = lax.rsqrt(deg)
    h = lax.dot_general(x_ref[...], w_ref[...],
                        (((1,), (1,)), ((), ())),
                        preferred_element_type=jnp.float32)
    g_ref[...] = dinv * (h + b_ref[...].reshape(1, D))
    dinv_ref[...] = dinv


_lin_call = pl.pallas_call(
    _lin_body,
    out_shape=[
        jax.ShapeDtypeStruct((N_PAD, D), jnp.float32),
        jax.ShapeDtypeStruct((N_PAD, 1), jnp.float32),
    ],
)


def _fin_body(s_ref, g_ref, dinv_ref, o_ref):
    s = s_ref[0] + s_ref[1] + g_ref[...]
    o_ref[...] = (COEF * jnp.maximum(C_U * dinv_ref[...] * s, 0.0))[:N]


_fin_call = pl.pallas_call(
    _fin_body,
    out_shape=jax.ShapeDtypeStruct((N, D), jnp.float32),
)


def kernel(x, edge_index, W, b):
    row = edge_index[0]
    col = edge_index[1]
    pad = (N + jnp.arange(E_PAD - E, dtype=jnp.int32) % (N_PAD - N))
    row_pad = jnp.concatenate([row, pad]).reshape(NW * NCHUNK, CHUNK)
    col_pad = jnp.concatenate([col, pad]).reshape(NW * NCHUNK, CHUNK)
    x_pad = jnp.pad(x, ((0, N_PAD - N), (0, 0)))

    degp = _deg_kernel(row_pad)
    g, dinv = _lin_call(x_pad, W, b, degp)

    zeros = jnp.zeros((ZROWS, D), jnp.float32)
    S = _scatter_kernel(g, row_pad, col_pad, zeros)
    return _fin_call(S, g, dinv)


# speedup vs baseline: 3.5738x; 1.0373x over previous
import functools

import numpy as np
import jax
import jax.numpy as jnp
from jax import lax
from jax.experimental import pallas as pl
from jax.experimental.pallas import tpu as pltpu
from jax.experimental.pallas import tpu_sc as plsc

N = 10000
E = 320000
D = 128
C_U = 1.0
C_SIGMA = 2.0
COEF = float(np.sqrt(C_SIGMA / D))

NC, NS = 2, 16
NW = NC * NS
CHUNK = 128
N_PAD = 10240
SLAB = N_PAD // NS
NCHUNK = 80
TILE_E = NCHUNK * CHUNK
E_PAD = NW * TILE_E
ZROWS = 128

_MESH = plsc.VectorSubcoreMesh(core_axis_name="c", subcore_axis_name="s")


@functools.partial(
    pl.kernel,
    out_type=jax.ShapeDtypeStruct((NC, N_PAD), jnp.float32),
    mesh=_MESH,
    scratch_types=[
        pltpu.VMEM_SHARED((N_PAD,), jnp.float32),
        pltpu.VMEM((NCHUNK, CHUNK), jnp.int32),
        pltpu.VMEM((CHUNK,), jnp.float32),
        pltpu.VMEM((SLAB,), jnp.float32),
    ],
)
def _deg_kernel(row_hbm, deg_out, deg_sh, row_v, ones_v, zb_v):
    cid = lax.axis_index("c")
    sid = lax.axis_index("s")
    wid = cid * NS + sid

    def fill_ones(i, _):
        ones_v[pl.ds(i * 16, 16)] = jnp.ones((16,), jnp.float32)
        return 0

    lax.fori_loop(0, CHUNK // 16, fill_ones, 0)

    def fill_zero(i, _):
        zb_v[pl.ds(i * 16, 16)] = jnp.zeros((16,), jnp.float32)
        return 0

    lax.fori_loop(0, SLAB // 16, fill_zero, 0)
    pltpu.sync_copy(zb_v, deg_sh.at[pl.ds(sid * SLAB, SLAB)])
    pltpu.sync_copy(row_hbm.at[pl.ds(wid * NCHUNK, NCHUNK)], row_v)
    plsc.subcore_barrier()

    def chunk(i, _):
        pltpu.sync_copy(ones_v, deg_sh.at[row_v.at[i]], add=True)
        return 0

    lax.fori_loop(0, NCHUNK, chunk, 0)
    plsc.subcore_barrier()
    pltpu.sync_copy(deg_sh.at[pl.ds(sid * SLAB, SLAB)],
                    deg_out.at[cid, pl.ds(sid * SLAB, SLAB)])


GROUP = 8
NGROUP = NCHUNK // GROUP


@functools.partial(
    pl.kernel,
    out_type=jax.ShapeDtypeStruct((NC, N_PAD, D), jnp.float32),
    mesh=_MESH,
    scratch_types=[
        pltpu.VMEM_SHARED((N_PAD, D), jnp.float32),
        pltpu.VMEM((GROUP, CHUNK), jnp.int32),
        pltpu.VMEM((GROUP, CHUNK), jnp.int32),
        pltpu.VMEM((CHUNK, D), jnp.float32),
        pltpu.VMEM((CHUNK, D), jnp.float32),
        pltpu.SemaphoreType.DMA,
        pltpu.SemaphoreType.DMA,
    ],
)
def _scatter_kernel(g_hbm, row_hbm, col_hbm, z_hbm, s_out,
                    acc_sh, row_v, col_v, buf_a, buf_b, sem_a, sem_b):
    cid = lax.axis_index("c")
    sid = lax.axis_index("s")
    wid = cid * NS + sid

    @pl.when(cid == 0)
    def _():
        pltpu.sync_copy(g_hbm.at[pl.ds(sid * SLAB, SLAB)],
                        acc_sh.at[pl.ds(sid * SLAB, SLAB)])

    @pl.when(cid != 0)
    def _():
        for k in range(SLAB // ZROWS):
            pltpu.sync_copy(z_hbm,
                            acc_sh.at[pl.ds(sid * SLAB + k * ZROWS, ZROWS)])

    plsc.subcore_barrier()

    bufs = (buf_a, buf_b)
    sems = (sem_a, sem_b)

    def group(gi, _):
        blk = wid * NCHUNK + gi * GROUP
        pltpu.sync_copy(row_hbm.at[pl.ds(blk, GROUP)], row_v)
        pltpu.sync_copy(col_hbm.at[pl.ds(blk, GROUP)], col_v)
        pltpu.async_copy(g_hbm.at[row_v.at[0]], buf_a, sem_a)
        for k in range(GROUP - 1):
            pltpu.async_copy(g_hbm.at[row_v.at[k + 1]],
                             bufs[(k + 1) % 2], sems[(k + 1) % 2])
            pltpu.make_async_copy(g_hbm.at[pl.ds(0, CHUNK)],
                                  bufs[k % 2], sems[k % 2]).wait()
            pltpu.sync_copy(bufs[k % 2], acc_sh.at[col_v.at[k]], add=True)
        pltpu.make_async_copy(g_hbm.at[pl.ds(0, CHUNK)],
                              bufs[(GROUP - 1) % 2], sems[(GROUP - 1) % 2]).wait()
        pltpu.sync_copy(bufs[(GROUP - 1) % 2],
                        acc_sh.at[col_v.at[GROUP - 1]], add=True)
        return 0

    lax.fori_loop(0, NGROUP, group, 0)
    plsc.subcore_barrier()
    pltpu.sync_copy(acc_sh.at[pl.ds(sid * SLAB, SLAB)],
                    s_out.at[cid, pl.ds(sid * SLAB, SLAB)])


def _lin_body(x_ref, w_ref, b_ref, dp_ref, g_ref, dinv_ref):
    deg = (dp_ref[0] + dp_ref[1] + 1.0).reshape(N_PAD, 1)
    dinv = lax.rsqrt(deg)
    h = lax.dot_general(x_ref[...], w_ref[...],
                        (((1,), (1,)), ((), ())),
                        preferred_element_type=jnp.float32)
    g_ref[pl.ds(0, N), :] = dinv[:N] * (h + b_ref[...].reshape(1, D))
    g_ref[pl.ds(N, N_PAD - N), :] = jnp.zeros((N_PAD - N, D), jnp.float32)
    dinv_ref[...] = dinv


_lin_call = pl.pallas_call(
    _lin_body,
    out_shape=[
        jax.ShapeDtypeStruct((N_PAD, D), jnp.float32),
        jax.ShapeDtypeStruct((N_PAD, 1), jnp.float32),
    ],
)


def _fin_body(s_ref, dinv_ref, o_ref):
    s = s_ref[0] + s_ref[1]
    o_ref[...] = (COEF * jnp.maximum(C_U * dinv_ref[...] * s, 0.0))[:N]


_fin_call = pl.pallas_call(
    _fin_body,
    out_shape=jax.ShapeDtypeStruct((N, D), jnp.float32),
)


def kernel(x, edge_index, W, b):
    row = edge_index[0]
    col = edge_index[1]
    pad = (N + jnp.arange(E_PAD - E, dtype=jnp.int32) % (N_PAD - N))
    row_pad = jnp.concatenate([row, pad]).reshape(NW * NCHUNK, CHUNK)
    col_pad = jnp.concatenate([col, pad]).reshape(NW * NCHUNK, CHUNK)

    degp = _deg_kernel(row_pad)
    g, dinv = _lin_call(x, W, b, degp)

    zeros = jnp.zeros((ZROWS, D), jnp.float32)
    S = _scatter_kernel(g, row_pad, col_pad, zeros)
    return _fin_call(S, dinv)


# speedup vs baseline: 3.6517x; 1.0218x over previous
import functools

import numpy as np
import jax
import jax.numpy as jnp
from jax import lax
from jax.experimental import pallas as pl
from jax.experimental.pallas import tpu as pltpu
from jax.experimental.pallas import tpu_sc as plsc

N = 10000
E = 320000
D = 128
C_U = 1.0
C_SIGMA = 2.0
COEF = float(np.sqrt(C_SIGMA / D))

NC, NS = 2, 16
NW = NC * NS
CHUNK = 128
N_PAD = 10240
SLAB = N_PAD // NS
NCHUNK = 80
TILE_E = NCHUNK * CHUNK
E_PAD = NW * TILE_E
ZROWS = 128

_MESH = plsc.VectorSubcoreMesh(core_axis_name="c", subcore_axis_name="s")


@functools.partial(
    pl.kernel,
    out_type=jax.ShapeDtypeStruct((NC, N_PAD), jnp.float32),
    mesh=_MESH,
    scratch_types=[
        pltpu.VMEM_SHARED((N_PAD,), jnp.float32),
        pltpu.VMEM((NCHUNK, CHUNK), jnp.int32),
        pltpu.VMEM((CHUNK,), jnp.float32),
        pltpu.VMEM((SLAB,), jnp.float32),
        pltpu.SemaphoreType.DMA,
    ],
)
def _deg_kernel(row_hbm, deg_out, deg_sh, row_v, ones_v, zb_v, sem_d):
    cid = lax.axis_index("c")
    sid = lax.axis_index("s")
    wid = cid * NS + sid

    def fill_ones(i, _):
        ones_v[pl.ds(i * 16, 16)] = jnp.ones((16,), jnp.float32)
        return 0

    lax.fori_loop(0, CHUNK // 16, fill_ones, 0)

    def fill_zero(i, _):
        zb_v[pl.ds(i * 16, 16)] = jnp.zeros((16,), jnp.float32)
        return 0

    lax.fori_loop(0, SLAB // 16, fill_zero, 0)
    pltpu.sync_copy(zb_v, deg_sh.at[pl.ds(sid * SLAB, SLAB)])
    pltpu.sync_copy(row_hbm.at[pl.ds(wid * NCHUNK, NCHUNK)], row_v)
    plsc.subcore_barrier()

    def chunk(gi, _):
        for k in range(8):
            pltpu.async_copy(ones_v, deg_sh.at[row_v.at[gi * 8 + k]],
                             sem_d, add=True)
        for k in range(8):
            pltpu.make_async_copy(ones_v, deg_sh.at[row_v.at[gi * 8 + k]],
                                  sem_d).wait()
        return 0

    lax.fori_loop(0, NCHUNK // 8, chunk, 0)
    plsc.subcore_barrier()
    pltpu.sync_copy(deg_sh.at[pl.ds(sid * SLAB, SLAB)],
                    deg_out.at[cid, pl.ds(sid * SLAB, SLAB)])


GROUP = 8
NGROUP = NCHUNK // GROUP


@functools.partial(
    pl.kernel,
    out_type=jax.ShapeDtypeStruct((NC, N_PAD, D), jnp.float32),
    mesh=_MESH,
    scratch_types=[
        pltpu.VMEM_SHARED((N_PAD, D), jnp.float32),
        pltpu.VMEM((GROUP, CHUNK), jnp.int32),
        pltpu.VMEM((GROUP, CHUNK), jnp.int32),
        pltpu.VMEM((CHUNK, D), jnp.float32),
        pltpu.VMEM((CHUNK, D), jnp.float32),
        pltpu.SemaphoreType.DMA,
        pltpu.SemaphoreType.DMA,
    ],
)
def _scatter_kernel(g_hbm, row_hbm, col_hbm, z_hbm, s_out,
                    acc_sh, row_v, col_v, buf_a, buf_b, sem_a, sem_b):
    cid = lax.axis_index("c")
    sid = lax.axis_index("s")
    wid = cid * NS + sid

    @pl.when(cid == 0)
    def _():
        pltpu.sync_copy(g_hbm.at[pl.ds(sid * SLAB, SLAB)],
                        acc_sh.at[pl.ds(sid * SLAB, SLAB)])

    @pl.when(cid != 0)
    def _():
        for k in range(SLAB // ZROWS):
            pltpu.sync_copy(z_hbm,
                            acc_sh.at[pl.ds(sid * SLAB + k * ZROWS, ZROWS)])

    plsc.subcore_barrier()

    bufs = (buf_a, buf_b)
    sems = (sem_a, sem_b)

    def group(gi, _):
        blk = wid * NCHUNK + gi * GROUP
        pltpu.sync_copy(row_hbm.at[pl.ds(blk, GROUP)], row_v)
        pltpu.sync_copy(col_hbm.at[pl.ds(blk, GROUP)], col_v)
        pltpu.async_copy(g_hbm.at[row_v.at[0]], buf_a, sem_a)
        for k in range(GROUP - 1):
            pltpu.async_copy(g_hbm.at[row_v.at[k + 1]],
                             bufs[(k + 1) % 2], sems[(k + 1) % 2])
            pltpu.make_async_copy(g_hbm.at[pl.ds(0, CHUNK)],
                                  bufs[k % 2], sems[k % 2]).wait()
            pltpu.sync_copy(bufs[k % 2], acc_sh.at[col_v.at[k]], add=True)
        pltpu.make_async_copy(g_hbm.at[pl.ds(0, CHUNK)],
                              bufs[(GROUP - 1) % 2], sems[(GROUP - 1) % 2]).wait()
        pltpu.sync_copy(bufs[(GROUP - 1) % 2],
                        acc_sh.at[col_v.at[GROUP - 1]], add=True)
        return 0

    lax.fori_loop(0, NGROUP, group, 0)
    plsc.subcore_barrier()
    pltpu.sync_copy(acc_sh.at[pl.ds(sid * SLAB, SLAB)],
                    s_out.at[cid, pl.ds(sid * SLAB, SLAB)])


def _lin_body(x_ref, w_ref, b_ref, dp_ref, g_ref, dinv_ref):
    deg = (dp_ref[0] + dp_ref[1] + 1.0).reshape(N_PAD, 1)
    dinv = lax.rsqrt(deg)
    h = lax.dot_general(x_ref[...], w_ref[...],
                        (((1,), (1,)), ((), ())),
                        preferred_element_type=jnp.float32)
    g_ref[pl.ds(0, N), :] = dinv[:N] * (h + b_ref[...].reshape(1, D))
    g_ref[pl.ds(N, N_PAD - N), :] = jnp.zeros((N_PAD - N, D), jnp.float32)
    dinv_ref[...] = dinv


_lin_call = pl.pallas_call(
    _lin_body,
    out_shape=[
        jax.ShapeDtypeStruct((N_PAD, D), jnp.float32),
        jax.ShapeDtypeStruct((N_PAD, 1), jnp.float32),
    ],
)


def _fin_body(s_ref, dinv_ref, o_ref):
    s = s_ref[0] + s_ref[1]
    o_ref[...] = (COEF * jnp.maximum(C_U * dinv_ref[...] * s, 0.0))[:N]


_fin_call = pl.pallas_call(
    _fin_body,
    out_shape=jax.ShapeDtypeStruct((N, D), jnp.float32),
)


def kernel(x, edge_index, W, b):
    row = edge_index[0]
    col = edge_index[1]
    pad = (N + jnp.arange(E_PAD - E, dtype=jnp.int32) % (N_PAD - N))
    row_pad = jnp.concatenate([row, pad]).reshape(NW * NCHUNK, CHUNK)
    col_pad = jnp.concatenate([col, pad]).reshape(NW * NCHUNK, CHUNK)

    degp = _deg_kernel(row_pad)
    g, dinv = _lin_call(x, W, b, degp)

    zeros = jnp.zeros((ZROWS, D), jnp.float32)
    S = _scatter_kernel(g, row_pad, col_pad, zeros)
    return _fin_call(S, dinv)
